# baseline (speedup 1.0000x reference)
import jax
import jax.numpy as jnp
from jax.experimental import pallas as pl

N = 64
N_IN = 128
N_H = 128
BB = 128


def _fused_gcn_kernel(adj_ref, seq_ref, w_ref, bias_ref, a_ref, out_ref):
    bb = out_ref.shape[0]
    seq = seq_ref[...].reshape(bb * N, N_IN)
    sf = jnp.dot(seq, w_ref[...], preferred_element_type=jnp.float32)
    sf = sf.reshape(bb, N, N_H)
    adj2 = adj_ref[...].reshape(bb, N // 2, 2 * N)
    a_even = adj2[:, :, :N]
    a_odd = adj2[:, :, N:]
    dn = (((2,), (1,)), ((0,), (0,)))
    out_e = jax.lax.dot_general(a_even, sf, dn, preferred_element_type=jnp.float32)
    out_o = jax.lax.dot_general(a_odd, sf, dn, preferred_element_type=jnp.float32)
    row = jax.lax.broadcasted_iota(jnp.int32, (bb, N // 2, N_H), 1)
    out_o = jnp.where(row < N // 2 - 1, out_o, jnp.inf)
    m = jnp.minimum(jnp.min(out_e, axis=1), jnp.min(out_o, axis=1))
    m = m + bias_ref[...]
    a = a_ref[0, 0]
    out_ref[...] = jnp.where(m >= 0, m, a * m)


def kernel(adj, seq1, W, bias, prelu_a):
    B = adj.shape[0]
    grid = (B // BB,)
    return pl.pallas_call(
        _fused_gcn_kernel,
        grid=grid,
        in_specs=[
            pl.BlockSpec((BB * N // 2, 2 * N), lambda i: (i, 0)),
            pl.BlockSpec((BB, N, N_IN), lambda i: (i, 0, 0)),
            pl.BlockSpec((N_IN, N_H), lambda i: (0, 0)),
            pl.BlockSpec((1, N_H), lambda i: (0, 0)),
            pl.BlockSpec((1, 1), lambda i: (0, 0)),
        ],
        out_specs=pl.BlockSpec((BB, N_H), lambda i: (i, 0)),
        out_shape=jax.ShapeDtypeStruct((B, N_H), jnp.float32),
    )(adj.reshape(B * N // 2, 2 * N), seq1, W,
      bias.reshape(1, N_H), prelu_a.reshape(1, 1))


# speedup vs baseline: 1.6106x; 1.6106x over previous
import jax
import jax.numpy as jnp
from jax.experimental import pallas as pl

N = 64
N_IN = 128
N_H = 128
BB = 128


def _fused_gcn_kernel(adj_ref, seq_ref, w_ref, bias_ref, a_ref, out_ref):
    bb = out_ref.shape[0]
    seq = seq_ref[...].reshape(bb * N, N_IN)
    sf = jnp.dot(seq, w_ref[...], preferred_element_type=jnp.float32)
    sf = sf.reshape(bb, N, N_H).astype(jnp.bfloat16)
    adj2 = adj_ref[...]
    a_even = adj2[:, :, :N]
    a_odd = adj2[:, :, N:]
    dn = (((2,), (1,)), ((0,), (0,)))
    out_e = jax.lax.dot_general(a_even, sf, dn, preferred_element_type=jnp.float32)
    out_o = jax.lax.dot_general(a_odd, sf, dn, preferred_element_type=jnp.float32)
    row = jax.lax.broadcasted_iota(jnp.int32, (bb, N // 2, N_H), 1)
    out_o = jnp.where(row < N // 2 - 1, out_o, jnp.inf)
    m = jnp.minimum(jnp.min(out_e, axis=1), jnp.min(out_o, axis=1))
    m = m + bias_ref[...]
    a = a_ref[0, 0]
    out_ref[...] = jnp.where(m >= 0, m, a * m)


def kernel(adj, seq1, W, bias, prelu_a):
    B = adj.shape[0]
    grid = (B // BB,)
    return pl.pallas_call(
        _fused_gcn_kernel,
        grid=grid,
        in_specs=[
            pl.BlockSpec((BB, N // 2, 2 * N), lambda i: (i, 0, 0)),
            pl.BlockSpec((BB, N, N_IN), lambda i: (i, 0, 0)),
            pl.BlockSpec((N_IN, N_H), lambda i: (0, 0)),
            pl.BlockSpec((1, N_H), lambda i: (0, 0)),
            pl.BlockSpec((1, 1), lambda i: (0, 0)),
        ],
        out_specs=pl.BlockSpec((BB, N_H), lambda i: (i, 0)),
        out_shape=jax.ShapeDtypeStruct((B, N_H), jnp.float32),
    )(adj.astype(jnp.bfloat16).reshape(B, N // 2, 2 * N), seq1, W,
      bias.reshape(1, N_H), prelu_a.reshape(1, 1))


# speedup vs baseline: 2.7465x; 1.7052x over previous
import jax
import jax.numpy as jnp
from jax.experimental import pallas as pl

N = 64
N_IN = 128
N_H = 128
BB = 128


def _fused_gcn_kernel(adj_ref, seq_ref, w_ref, bias_ref, a_ref, out_ref):
    bb = out_ref.shape[0]
    seq = seq_ref[...].reshape(bb * N, N_IN)
    sf = jnp.dot(seq, w_ref[...], preferred_element_type=jnp.float32)
    sf = sf.reshape(bb, N, N_H)
    adj_t = jnp.transpose(adj_ref[...], (2, 0, 1))
    out = jax.lax.dot_general(
        adj_t, sf,
        dimension_numbers=(((2,), (1,)), ((0,), (0,))),
        preferred_element_type=jnp.float32,
    )
    row = jax.lax.broadcasted_iota(jnp.int32, (bb, N, N_H), 1)
    out = jnp.where(row < N - 1, out, jnp.inf)
    m = jnp.min(out, axis=1) + bias_ref[...]
    a = a_ref[0, 0]
    out_ref[...] = jnp.where(m >= 0, m, a * m)


def kernel(adj, seq1, W, bias, prelu_a):
    B = adj.shape[0]
    grid = (B // BB,)
    return pl.pallas_call(
        _fused_gcn_kernel,
        grid=grid,
        in_specs=[
            pl.BlockSpec((N, N, BB), lambda i: (0, 0, i)),
            pl.BlockSpec((BB, N, N_IN), lambda i: (i, 0, 0)),
            pl.BlockSpec((N_IN, N_H), lambda i: (0, 0)),
            pl.BlockSpec((1, N_H), lambda i: (0, 0)),
            pl.BlockSpec((1, 1), lambda i: (0, 0)),
        ],
        out_specs=pl.BlockSpec((BB, N_H), lambda i: (i, 0)),
        out_shape=jax.ShapeDtypeStruct((B, N_H), jnp.float32),
    )(adj.transpose(1, 2, 0), seq1, W,
      bias.reshape(1, N_H), prelu_a.reshape(1, 1))


# speedup vs baseline: 2.9537x; 1.0755x over previous
import jax
import jax.numpy as jnp
from jax.experimental import pallas as pl

N = 64
N_IN = 128
N_H = 128
BB = 128


def _fused_gcn_kernel(adj_ref, seq_ref, w_ref, bias_ref, a_ref, out_ref):
    bb = out_ref.shape[0]
    seq = seq_ref[...].reshape(bb * N, N_IN)
    sf = jnp.dot(seq, w_ref[...], preferred_element_type=jnp.float32)
    sf = sf.reshape(bb, N, N_H).astype(jnp.bfloat16)
    out = jax.lax.dot_general(
        adj_ref[...].astype(jnp.bfloat16), sf,
        dimension_numbers=(((1,), (1,)), ((2,), (0,))),
        preferred_element_type=jnp.float32,
    )
    row = jax.lax.broadcasted_iota(jnp.int32, (bb, N, N_H), 1)
    out = jnp.where(row < N - 1, out, jnp.inf)
    m = jnp.min(out, axis=1) + bias_ref[...]
    a = a_ref[0, 0]
    out_ref[...] = jnp.where(m >= 0, m, a * m)


def kernel(adj, seq1, W, bias, prelu_a):
    B = adj.shape[0]
    grid = (B // BB,)
    return pl.pallas_call(
        _fused_gcn_kernel,
        grid=grid,
        in_specs=[
            pl.BlockSpec((N, N, BB), lambda i: (0, 0, i)),
            pl.BlockSpec((BB, N, N_IN), lambda i: (i, 0, 0)),
            pl.BlockSpec((N_IN, N_H), lambda i: (0, 0)),
            pl.BlockSpec((1, N_H), lambda i: (0, 0)),
            pl.BlockSpec((1, 1), lambda i: (0, 0)),
        ],
        out_specs=pl.BlockSpec((BB, N_H), lambda i: (i, 0)),
        out_shape=jax.ShapeDtypeStruct((B, N_H), jnp.float32),
    )(adj.transpose(1, 2, 0), seq1, W,
      bias.reshape(1, N_H), prelu_a.reshape(1, 1))


# speedup vs baseline: 3.4648x; 1.1730x over previous
import jax
import jax.numpy as jnp
from jax.experimental import pallas as pl

N = 64
N_IN = 128
N_H = 128
BB = 256


def _fused_gcn_kernel(adj_ref, seq_ref, w_ref, bias_ref, a_ref, out_ref):
    bb = out_ref.shape[0]
    seq = seq_ref[...].reshape(bb * N, N_IN)
    sf = jnp.dot(seq, w_ref[...], preferred_element_type=jnp.float32)
    sf = sf.reshape(bb, N, N_H).astype(jnp.bfloat16)
    out = jax.lax.dot_general(
        adj_ref[...].astype(jnp.bfloat16), sf,
        dimension_numbers=(((1,), (1,)), ((2,), (0,))),
        preferred_element_type=jnp.float32,
    )
    m1 = jnp.min(out[:, : N - 8, :], axis=1)
    tail = out[:, N - 8 :, :]
    row = jax.lax.broadcasted_iota(jnp.int32, (bb, 8, N_H), 1)
    tail = jnp.where(row < 7, tail, jnp.inf)
    m = jnp.minimum(m1, jnp.min(tail, axis=1)) + bias_ref[...]
    a = a_ref[0, 0]
    out_ref[...] = jnp.where(m >= 0, m, a * m)


def kernel(adj, seq1, W, bias, prelu_a):
    B = adj.shape[0]
    grid = (B // BB,)
    return pl.pallas_call(
        _fused_gcn_kernel,
        grid=grid,
        in_specs=[
            pl.BlockSpec((N, N, BB), lambda i: (0, 0, i)),
            pl.BlockSpec((BB, N, N_IN), lambda i: (i, 0, 0)),
            pl.BlockSpec((N_IN, N_H), lambda i: (0, 0)),
            pl.BlockSpec((1, N_H), lambda i: (0, 0)),
            pl.BlockSpec((1, 1), lambda i: (0, 0)),
        ],
        out_specs=pl.BlockSpec((BB, N_H), lambda i: (i, 0)),
        out_shape=jax.ShapeDtypeStruct((B, N_H), jnp.float32),
    )(adj.transpose(1, 2, 0), seq1, W,
      bias.reshape(1, N_H), prelu_a.reshape(1, 1))


# speedup vs baseline: 3.5473x; 1.0238x over previous
import jax
import jax.numpy as jnp
from jax.experimental import pallas as pl

N = 64
N_IN = 128
N_H = 128
BB = 256


def _fused_gcn_kernel(adj_ref, seq_ref, w_ref, bias_ref, a_ref, out_ref):
    bb = out_ref.shape[0]
    seq = seq_ref[...].reshape(bb * N, N_IN)
    sf = jnp.dot(seq, w_ref[...], preferred_element_type=jnp.float32)
    sf = sf.reshape(bb, N, N_H).astype(jnp.bfloat16)
    out = jax.lax.dot_general(
        adj_ref[...].astype(jnp.bfloat16), sf,
        dimension_numbers=(((1,), (1,)), ((2,), (0,))),
        preferred_element_type=jnp.float32,
    )
    m = jnp.min(out, axis=1) + bias_ref[...]
    a = a_ref[0, 0]
    out_ref[...] = jnp.where(m >= 0, m, a * m)


def kernel(adj, seq1, W, bias, prelu_a):
    B = adj.shape[0]
    grid = (B // BB,)
    return pl.pallas_call(
        _fused_gcn_kernel,
        grid=grid,
        in_specs=[
            pl.BlockSpec((N - 1, N, BB), lambda i: (0, 0, i)),
            pl.BlockSpec((BB, N, N_IN), lambda i: (i, 0, 0)),
            pl.BlockSpec((N_IN, N_H), lambda i: (0, 0)),
            pl.BlockSpec((1, N_H), lambda i: (0, 0)),
            pl.BlockSpec((1, 1), lambda i: (0, 0)),
        ],
        out_specs=pl.BlockSpec((BB, N_H), lambda i: (i, 0)),
        out_shape=jax.ShapeDtypeStruct((B, N_H), jnp.float32),
    )(adj.transpose(1, 2, 0), seq1, W,
      bias.reshape(1, N_H), prelu_a.reshape(1, 1))
